# Initial kernel scaffold; baseline (speedup 1.0000x reference)
import functools

import jax
import jax.numpy as jnp
from jax import lax
from jax.experimental import pallas as pl
from jax.experimental.pallas import tpu as pltpu
from jax.experimental.pallas import tpu_sc as plsc

D = 32
L = 128
G = 8
NW = 32
B = 16384 * 26
PER_W = B // NW
CHUNKS = PER_W // L
GROUPS = CHUNKS // G


def _gather_call(idx, table):
    mesh = plsc.VectorSubcoreMesh(core_axis_name="c", subcore_axis_name="s")

    @functools.partial(
        pl.kernel,
        mesh=mesh,
        out_type=jax.ShapeDtypeStruct((NW, GROUPS, G, L, D), jnp.float32),
        scratch_types=[
            pltpu.VMEM((CHUNKS, L), jnp.int32),
            pltpu.VMEM((G, L, D), jnp.float32),
            pltpu.SemaphoreType.DMA,
        ],
    )
    def k(idx_hbm, table_hbm, out_hbm, idx_v, rows_v, sem):
        wid = lax.axis_index("s") * 2 + lax.axis_index("c")
        pltpu.sync_copy(idx_hbm.at[wid], idx_v)

        def body(g, carry):
            descs = [
                pltpu.async_copy(
                    table_hbm.at[idx_v.at[g * G + j]], rows_v.at[j], sem
                )
                for j in range(G)
            ]
            for d_ in descs:
                d_.wait()
            pltpu.sync_copy(rows_v, out_hbm.at[wid, g])
            return carry

        lax.fori_loop(0, GROUPS, body, 0)

    return k(idx, table)


def kernel(inputs, embeddings):
    idx = inputs.astype(jnp.int32).reshape(NW, CHUNKS, L)
    out = _gather_call(idx, embeddings)
    return out.reshape(16384, 26, D)


# speedup vs baseline: 1.5652x; 1.5652x over previous
import functools

import jax
import jax.numpy as jnp
from jax import lax
from jax.experimental import pallas as pl
from jax.experimental.pallas import tpu as pltpu
from jax.experimental.pallas import tpu_sc as plsc

D = 32
L = 128
G = 8
NW = 32
B = 16384 * 26
PER_W = B // NW
CHUNKS = PER_W // L
GROUPS = CHUNKS // G


def _gather_call(idx, table):
    mesh = plsc.VectorSubcoreMesh(core_axis_name="c", subcore_axis_name="s")

    @functools.partial(
        pl.kernel,
        mesh=mesh,
        out_type=jax.ShapeDtypeStruct((NW, GROUPS, G, L, D), jnp.float32),
        scratch_types=[
            pltpu.VMEM((CHUNKS, L), jnp.int32),
            pltpu.VMEM((G, L, D), jnp.float32),
            pltpu.SemaphoreType.DMA,
        ],
        compiler_params=pltpu.CompilerParams(use_tc_tiling_on_sc=False),
    )
    def k(idx_hbm, table_hbm, out_hbm, idx_v, rows_v, sem):
        wid = lax.axis_index("s") * 2 + lax.axis_index("c")
        pltpu.sync_copy(idx_hbm.at[wid], idx_v)

        def body(g, carry):
            descs = [
                pltpu.async_copy(
                    table_hbm.at[idx_v.at[g * G + j]], rows_v.at[j], sem
                )
                for j in range(G)
            ]
            for d_ in descs:
                d_.wait()
            pltpu.sync_copy(rows_v, out_hbm.at[wid, g])
            return carry

        lax.fori_loop(0, GROUPS, body, 0)

    return k(idx, table)


def kernel(inputs, embeddings):
    idx = inputs.astype(jnp.int32).reshape(NW, CHUNKS, L)
    out = _gather_call(idx, embeddings)
    return out.reshape(16384, 26, D)


# speedup vs baseline: 1.5669x; 1.0011x over previous
import functools

import jax
import jax.numpy as jnp
from jax import lax
from jax.experimental import pallas as pl
from jax.experimental.pallas import tpu as pltpu
from jax.experimental.pallas import tpu_sc as plsc

BATCH = 16384
SEQ = 26
D = 32
NW = 32
PER_B = BATCH // NW
NB = 32
NCH = PER_B // NB


def _gather_call(idx, table):
    mesh = plsc.VectorSubcoreMesh(core_axis_name="c", subcore_axis_name="s")

    @functools.partial(
        pl.kernel,
        mesh=mesh,
        out_type=jax.ShapeDtypeStruct((BATCH, SEQ, D), jnp.float32),
        scratch_types=[
            pltpu.VMEM((PER_B, SEQ), jnp.int32),
            pltpu.VMEM((NB, SEQ, D), jnp.float32),
            pltpu.VMEM((NB, SEQ, D), jnp.float32),
            pltpu.SemaphoreType.DMA,
            pltpu.SemaphoreType.DMA,
        ],
        compiler_params=pltpu.CompilerParams(use_tc_tiling_on_sc=False),
    )
    def k(idx_hbm, table_hbm, out_hbm, idx_v, buf_a, buf_b, sem_a, sem_b):
        wid = lax.axis_index("s") * 2 + lax.axis_index("c")
        b0 = wid * PER_B
        pltpu.sync_copy(idx_hbm.at[pl.ds(b0, PER_B)], idx_v)

        bufs = (buf_a, buf_b)
        sems = (sem_a, sem_b)

        def fire(c, buf, sem):
            def body(i, carry):
                pltpu.async_copy(
                    table_hbm.at[idx_v.at[c * NB + i]], buf.at[i], sem
                )
                return carry

            lax.fori_loop(0, NB, body, 0)

        fire(0, bufs[0], sems[0])
        for c in range(NCH):
            buf, sem = bufs[c % 2], sems[c % 2]
            if c + 1 < NCH:
                fire(c + 1, bufs[(c + 1) % 2], sems[(c + 1) % 2])
            dst = out_hbm.at[pl.ds(b0 + c * NB, NB)]
            pltpu.make_async_copy(dst, buf, sem).wait()
            pltpu.sync_copy(buf, dst)

    return k(idx, table)


def kernel(inputs, embeddings):
    return _gather_call(inputs.astype(jnp.int32), embeddings)
